# baseline (speedup 1.0000x reference)
import jax
import jax.numpy as jnp
from jax import lax
from jax.experimental import pallas as pl
from jax.experimental.pallas import tpu as pltpu

HID = 768
INTER = 1024
NUM_ROUTED = 24
NUM_SHARED = 2
TOP_K = 6
RINTER = 256
N_SHARED_CHUNK = NUM_SHARED * (INTER // RINTER)
N_CHUNK = NUM_ROUTED + N_SHARED_CHUNK
QUAD = 4
N_STEP = N_CHUNK // QUAD
N_ROUTED_STEP = NUM_ROUTED // QUAD


def _moe_kernel(
    x_ref,
    gw_ref,
    rb2_ref,
    sb2_ref,
    rw1_ref,
    rw2_ref,
    sw1_ref,
    sw2_ref,
    cb1_ref,
    out_ref,
    w_scr,
    h_scr,
):
    g = pl.program_id(0)

    @pl.when(g == 0)
    def _():
        logits = lax.dot_general(
            x_ref[...],
            gw_ref[...],
            (((1,), (1,)), ((), ())),
            preferred_element_type=jnp.float32,
        )
        scores = jax.nn.sigmoid(logits)
        n, e = scores.shape
        col = lax.broadcasted_iota(jnp.int32, (n, e), 1)
        s = scores
        mask = jnp.zeros(scores.shape, dtype=jnp.bool_)
        for _ in range(TOP_K):
            m = jnp.max(s, axis=1, keepdims=True)
            is_max = s == m
            min_idx = jnp.min(jnp.where(is_max, col, e), axis=1, keepdims=True)
            pick = col == min_idx
            mask = mask | pick
            s = jnp.where(pick, -jnp.inf, s)
        sel = jnp.where(mask, scores, 0.0)
        w = sel / (jnp.sum(sel, axis=1, keepdims=True) + 1e-9)
        w_full = jnp.concatenate(
            [w, jnp.ones((n, N_SHARED_CHUNK), jnp.float32)], axis=1
        )
        for gg in range(N_STEP):
            w_scr[gg] = w_full[:, gg * QUAD : (gg + 1) * QUAD]
        shared_b2 = jnp.sum(sb2_ref[...], axis=0, keepdims=True)
        out_ref[...] = (
            jnp.dot(w, rb2_ref[...], preferred_element_type=jnp.float32) + shared_b2
        )

    routed = g < N_ROUTED_STEP

    @pl.when(routed)
    def _():
        wq = w_scr[g]
        w1 = jnp.concatenate([rw1_ref[i] for i in range(QUAD)], axis=1)
        hb = jnp.dot(x_ref[...], w1, preferred_element_type=jnp.float32)
        hb = jax.nn.gelu(hb + cb1_ref[0])
        for i in range(QUAD):
            sl = slice(i * RINTER, (i + 1) * RINTER)
            h_scr[:, sl] = hb[:, sl] * wq[:, i : i + 1]
        out_ref[...] += jnp.dot(
            h_scr[...], rw2_ref[...].reshape(INTER, HID),
            preferred_element_type=jnp.float32,
        )

    @pl.when(jnp.logical_not(routed))
    def _():
        hb = jnp.dot(x_ref[...], sw1_ref[0], preferred_element_type=jnp.float32)
        h_scr[...] = jax.nn.gelu(hb + cb1_ref[0])
        out_ref[...] += jnp.dot(
            h_scr[...], sw2_ref[0], preferred_element_type=jnp.float32
        )


def kernel(x, gate_W, sW1, sb1, sW2, sb2, rW1, rb1, rW2, rb2):
    b, s, d = x.shape
    xf = x.reshape(-1, d)
    n = xf.shape[0]

    sb1c = sb1.reshape(N_SHARED_CHUNK, RINTER)
    cb1 = jnp.concatenate([rb1, sb1c], axis=0).reshape(N_STEP, 1, INTER)

    def routed_idx(g):
        return (jnp.minimum(g, N_ROUTED_STEP - 1), 0, 0)

    def shared_idx(g):
        return (jnp.maximum(g - N_ROUTED_STEP, 0), 0, 0)

    out = pl.pallas_call(
        _moe_kernel,
        grid=(N_STEP,),
        in_specs=[
            pl.BlockSpec((n, HID), lambda g: (0, 0)),
            pl.BlockSpec((NUM_ROUTED, HID), lambda g: (0, 0)),
            pl.BlockSpec((NUM_ROUTED, HID), lambda g: (0, 0)),
            pl.BlockSpec((NUM_SHARED, HID), lambda g: (0, 0)),
            pl.BlockSpec((QUAD, HID, RINTER), routed_idx),
            pl.BlockSpec((QUAD, RINTER, HID), routed_idx),
            pl.BlockSpec((1, HID, INTER), shared_idx),
            pl.BlockSpec((1, INTER, HID), shared_idx),
            pl.BlockSpec((1, 1, INTER), lambda g: (g, 0, 0)),
        ],
        out_specs=pl.BlockSpec((n, HID), lambda g: (0, 0)),
        out_shape=jax.ShapeDtypeStruct((n, HID), jnp.float32),
        scratch_shapes=[
            pltpu.VMEM((N_STEP, n, QUAD), jnp.float32),
            pltpu.VMEM((n, INTER), jnp.float32),
        ],
        compiler_params=pltpu.CompilerParams(vmem_limit_bytes=100 * 1024 * 1024),
    )(xf, gate_W, rb2, sb2, rW1, rW2, sW1, sW2, cb1)

    aux_loss = jnp.asarray(0.0, dtype=jnp.float32)
    return (out.reshape(b, s, d), aux_loss)


# speedup vs baseline: 1.0055x; 1.0055x over previous
import jax
import jax.numpy as jnp
from jax import lax
from jax.experimental import pallas as pl
from jax.experimental.pallas import tpu as pltpu

HID = 768
INTER = 1024
NUM_ROUTED = 24
NUM_SHARED = 2
TOP_K = 6
RINTER = 256
N_SHARED_CHUNK = NUM_SHARED * (INTER // RINTER)
N_CHUNK = NUM_ROUTED + N_SHARED_CHUNK
QUAD = 4
N_STEP = N_CHUNK // QUAD
N_ROUTED_STEP = NUM_ROUTED // QUAD


def _moe_kernel(
    x_ref,
    gw_ref,
    rb2_ref,
    sb2_ref,
    rw1_ref,
    rw2_ref,
    sw1_ref,
    sw2_ref,
    cb1_ref,
    out_ref,
    w_scr,
):
    g = pl.program_id(0)

    @pl.when(g == 0)
    def _():
        logits = lax.dot_general(
            x_ref[...],
            gw_ref[...],
            (((1,), (1,)), ((), ())),
            preferred_element_type=jnp.float32,
        )
        scores = jax.nn.sigmoid(logits)
        n, e = scores.shape
        col = lax.broadcasted_iota(jnp.int32, (n, e), 1)
        s = scores
        mask = jnp.zeros(scores.shape, dtype=jnp.bool_)
        for _ in range(TOP_K):
            m = jnp.max(s, axis=1, keepdims=True)
            is_max = s == m
            min_idx = jnp.min(jnp.where(is_max, col, e), axis=1, keepdims=True)
            pick = col == min_idx
            mask = mask | pick
            s = jnp.where(pick, -jnp.inf, s)
        sel = jnp.where(mask, scores, 0.0)
        w = sel / (jnp.sum(sel, axis=1, keepdims=True) + 1e-9)
        w_full = jnp.concatenate(
            [w, jnp.ones((n, N_SHARED_CHUNK), jnp.float32)], axis=1
        )
        for gg in range(N_STEP):
            w_scr[gg] = w_full[:, gg * QUAD : (gg + 1) * QUAD]
        shared_b2 = jnp.sum(sb2_ref[...], axis=0, keepdims=True)
        out_ref[...] = (
            jnp.dot(w, rb2_ref[...], preferred_element_type=jnp.float32) + shared_b2
        )

    routed = g < N_ROUTED_STEP
    wq = w_scr[g]
    h_cols = []
    for i in range(QUAD):
        sl = slice(i * RINTER, (i + 1) * RINTER)
        w1_i = jnp.where(routed, rw1_ref[i], sw1_ref[0][:, sl])
        h_i = jnp.dot(x_ref[...], w1_i, preferred_element_type=jnp.float32)
        h_i = jax.nn.gelu(h_i + cb1_ref[0][:, sl]) * wq[:, i : i + 1]
        h_cols.append(h_i)
    h = jnp.concatenate(h_cols, axis=1)
    w2 = jnp.where(routed, rw2_ref[...].reshape(INTER, HID), sw2_ref[0])
    out_ref[...] += jnp.dot(h, w2, preferred_element_type=jnp.float32)


def kernel(x, gate_W, sW1, sb1, sW2, sb2, rW1, rb1, rW2, rb2):
    b, s, d = x.shape
    xf = x.reshape(-1, d)
    n = xf.shape[0]

    sb1c = sb1.reshape(N_SHARED_CHUNK, RINTER)
    cb1 = jnp.concatenate([rb1, sb1c], axis=0).reshape(N_STEP, 1, INTER)

    def routed_idx(g):
        return (jnp.minimum(g, N_ROUTED_STEP - 1), 0, 0)

    def shared_idx(g):
        return (jnp.maximum(g - N_ROUTED_STEP, 0), 0, 0)

    out = pl.pallas_call(
        _moe_kernel,
        grid=(N_STEP,),
        in_specs=[
            pl.BlockSpec((n, HID), lambda g: (0, 0)),
            pl.BlockSpec((NUM_ROUTED, HID), lambda g: (0, 0)),
            pl.BlockSpec((NUM_ROUTED, HID), lambda g: (0, 0)),
            pl.BlockSpec((NUM_SHARED, HID), lambda g: (0, 0)),
            pl.BlockSpec((QUAD, HID, RINTER), routed_idx),
            pl.BlockSpec((QUAD, RINTER, HID), routed_idx),
            pl.BlockSpec((1, HID, INTER), shared_idx),
            pl.BlockSpec((1, INTER, HID), shared_idx),
            pl.BlockSpec((1, 1, INTER), lambda g: (g, 0, 0)),
        ],
        out_specs=pl.BlockSpec((n, HID), lambda g: (0, 0)),
        out_shape=jax.ShapeDtypeStruct((n, HID), jnp.float32),
        scratch_shapes=[pltpu.VMEM((N_STEP, n, QUAD), jnp.float32)],
        compiler_params=pltpu.CompilerParams(vmem_limit_bytes=100 * 1024 * 1024),
    )(xf, gate_W, rb2, sb2, rW1, rW2, sW1, sW2, cb1)

    aux_loss = jnp.asarray(0.0, dtype=jnp.float32)
    return (out.reshape(b, s, d), aux_loss)
